# Initial kernel scaffold; baseline (speedup 1.0000x reference)
import functools

import jax
import jax.numpy as jnp
from jax import lax
from jax.experimental import pallas as pl
from jax.experimental.pallas import tpu as pltpu
from jax.experimental.pallas import tpu_sc as plsc

NC = 2
NS = 16
NW = NC * NS
K = 80
NPAD = 10240
SROWS = NPAD // NS
ZCHUNK = 80


def _sc_edge_aggregate(table, src3, dst3, n_chunks, d):
    mesh = plsc.VectorSubcoreMesh(core_axis_name="c", subcore_axis_name="s")

    @functools.partial(
        pl.kernel,
        out_type=jax.ShapeDtypeStruct((NC, NPAD, d), jnp.float32),
        mesh=mesh,
        scratch_types=[
            pltpu.VMEM((n_chunks, K), jnp.int32),
            pltpu.VMEM((n_chunks, K), jnp.int32),
            pltpu.VMEM((K, d), jnp.float32),
            pltpu.VMEM((ZCHUNK, d), jnp.float32),
            pltpu.VMEM_SHARED((NPAD, d), jnp.float32),
        ],
    )
    def agg_kernel(table_hbm, src_hbm, dst_hbm, out_hbm,
                   src_v, dst_v, rows_v, zb_v, acc_sh):
        cid = lax.axis_index("c")
        sid = lax.axis_index("s")
        wid = sid * NC + cid

        zvec = jnp.zeros((16,), jnp.float32)

        @pl.loop(0, ZCHUNK)
        def _(r):
            @pl.loop(0, d, step=16)
            def _(col):
                zb_v[r, pl.ds(col, 16)] = zvec

        @pl.loop(0, SROWS, step=ZCHUNK)
        def _(r0):
            pltpu.sync_copy(zb_v, acc_sh.at[pl.ds(sid * SROWS + r0, ZCHUNK)])

        plsc.subcore_barrier()

        pltpu.sync_copy(src_hbm.at[wid], src_v)
        pltpu.sync_copy(dst_hbm.at[wid], dst_v)

        @pl.loop(0, n_chunks)
        def _(j):
            pltpu.sync_copy(table_hbm.at[src_v.at[j]], rows_v)
            pltpu.sync_copy(rows_v, acc_sh.at[dst_v.at[j]], add=True)

        plsc.subcore_barrier()

        pltpu.sync_copy(acc_sh.at[pl.ds(sid * SROWS, SROWS)],
                        out_hbm.at[cid, pl.ds(sid * SROWS, SROWS)])

    return agg_kernel(table, src3, dst3)


def _tc_mlp(acc, w1, b1_2d, w2p, bm=1024):
    d_in = acc.shape[2]
    d_hid = w1.shape[1]
    d_out = w2p.shape[1]

    def body(a0, a1, w1r, b1r, w2r, o):
        agg = a0[0] + a1[0]
        h = jnp.dot(agg, w1r[...], preferred_element_type=jnp.float32)
        h = jnp.maximum(h + b1r[...], 0.0)
        o[...] = jnp.dot(h, w2r[...], preferred_element_type=jnp.float32)

    return pl.pallas_call(
        body,
        grid=(NPAD // bm,),
        in_specs=[
            pl.BlockSpec((1, bm, d_in), lambda i: (0, i, 0)),
            pl.BlockSpec((1, bm, d_in), lambda i: (1, i, 0)),
            pl.BlockSpec((d_in, d_hid), lambda i: (0, 0)),
            pl.BlockSpec((1, d_hid), lambda i: (0, 0)),
            pl.BlockSpec((d_hid, d_out), lambda i: (0, 0)),
        ],
        out_specs=pl.BlockSpec((bm, d_out), lambda i: (i, 0)),
        out_shape=jax.ShapeDtypeStruct((NPAD, d_out), jnp.float32),
    )(acc, acc, w1, b1_2d, w2p)


def _tc_final(acc, b2_2d, bm=1024):
    d = acc.shape[2]

    def body(a0, a1, b2r, o):
        o[...] = a0[0] + a1[0] + b2r[...]

    return pl.pallas_call(
        body,
        grid=(NPAD // bm,),
        in_specs=[
            pl.BlockSpec((1, bm, d), lambda i: (0, i, 0)),
            pl.BlockSpec((1, bm, d), lambda i: (1, i, 0)),
            pl.BlockSpec((1, d), lambda i: (0, 0)),
        ],
        out_specs=pl.BlockSpec((bm, d), lambda i: (i, 0)),
        out_shape=jax.ShapeDtypeStruct((NPAD, d), jnp.float32),
    )(acc, acc, b2_2d)


def kernel(x, edge_index, W1, b1, W2, b2):
    n_nodes, d_in = x.shape
    n_edges = edge_index.shape[1]
    d_hid = W1.shape[1]
    d_out = W2.shape[1]
    d_out_pad = 48

    ei = edge_index.astype(jnp.int32)
    n_chunks = n_edges // (NW * K)
    src3 = ei[0].reshape(NW, n_chunks, K)
    dst3 = ei[1].reshape(NW, n_chunks, K)

    acc1 = _sc_edge_aggregate(x, src3, dst3, n_chunks, d_in)

    w2p = jnp.pad(W2, ((0, 0), (0, d_out_pad - d_out)))
    g = _tc_mlp(acc1, W1, b1.reshape(1, d_hid), w2p)

    acc2 = _sc_edge_aggregate(g, src3, dst3, n_chunks, d_out_pad)

    b2p = jnp.pad(b2, (0, d_out_pad - d_out)).reshape(1, d_out_pad)
    out = _tc_final(acc2, b2p)
    return out[:n_nodes, :d_out]


# speedup vs baseline: 10.7736x; 10.7736x over previous
import functools

import jax
import jax.numpy as jnp
from jax import lax
from jax.experimental import pallas as pl
from jax.experimental.pallas import tpu as pltpu
from jax.experimental.pallas import tpu_sc as plsc

NC = 2
NS = 16
NW = NC * NS
K = 80
NPAD = 10240
SROWS = NPAD // NS
ZCHUNK = 80


def _zero_accumulator(zb_v, acc_sh, sid, d):
    zvec = jnp.zeros((16,), jnp.float32)

    @pl.loop(0, ZCHUNK)
    def _(r):
        @pl.loop(0, d, step=16)
        def _(col):
            zb_v[r, pl.ds(col, 16)] = zvec

    @pl.loop(0, SROWS, step=ZCHUNK)
    def _(r0):
        pltpu.sync_copy(zb_v, acc_sh.at[pl.ds(sid * SROWS + r0, ZCHUNK)])


def _agg_loop(table_hbm, src_v, dst_v, rows_v, acc_sh, n_chunks):
    @pl.loop(0, n_chunks)
    def _(j):
        pltpu.sync_copy(table_hbm.at[src_v.at[j]], rows_v)
        pltpu.sync_copy(rows_v, acc_sh.at[dst_v.at[j]], add=True)


def _sc_agg_featsplit(table0, table1, src3, dst3, n_chunks, d):
    mesh = plsc.VectorSubcoreMesh(core_axis_name="c", subcore_axis_name="s")

    @functools.partial(
        pl.kernel,
        out_type=jax.ShapeDtypeStruct((NC, NPAD, d), jnp.float32),
        mesh=mesh,
        scratch_types=[
            pltpu.VMEM((n_chunks, K), jnp.int32),
            pltpu.VMEM((n_chunks, K), jnp.int32),
            pltpu.VMEM((K, d), jnp.float32),
            pltpu.VMEM((ZCHUNK, d), jnp.float32),
            pltpu.VMEM_SHARED((NPAD, d), jnp.float32),
        ],
        compiler_params=pltpu.CompilerParams(use_tc_tiling_on_sc=False),
    )
    def agg_kernel(t0_hbm, t1_hbm, src_hbm, dst_hbm, out_hbm,
                   src_v, dst_v, rows_v, zb_v, acc_sh):
        cid = lax.axis_index("c")
        sid = lax.axis_index("s")

        _zero_accumulator(zb_v, acc_sh, sid, d)
        plsc.subcore_barrier()

        pltpu.sync_copy(src_hbm.at[sid], src_v)
        pltpu.sync_copy(dst_hbm.at[sid], dst_v)

        @pl.when(cid == 0)
        def _():
            _agg_loop(t0_hbm, src_v, dst_v, rows_v, acc_sh, n_chunks)

        @pl.when(cid == 1)
        def _():
            _agg_loop(t1_hbm, src_v, dst_v, rows_v, acc_sh, n_chunks)

        plsc.subcore_barrier()
        pltpu.sync_copy(acc_sh.at[pl.ds(sid * SROWS, SROWS)],
                        out_hbm.at[cid, pl.ds(sid * SROWS, SROWS)])

    return agg_kernel(table0, table1, src3, dst3)


def _sc_agg_edgesplit(table, src3, dst3, n_chunks, d):
    mesh = plsc.VectorSubcoreMesh(core_axis_name="c", subcore_axis_name="s")

    @functools.partial(
        pl.kernel,
        out_type=jax.ShapeDtypeStruct((NC, NPAD, d), jnp.float32),
        mesh=mesh,
        scratch_types=[
            pltpu.VMEM((n_chunks, K), jnp.int32),
            pltpu.VMEM((n_chunks, K), jnp.int32),
            pltpu.VMEM((K, d), jnp.float32),
            pltpu.VMEM((ZCHUNK, d), jnp.float32),
            pltpu.VMEM_SHARED((NPAD, d), jnp.float32),
        ],
        compiler_params=pltpu.CompilerParams(use_tc_tiling_on_sc=False),
    )
    def agg_kernel(table_hbm, src_hbm, dst_hbm, out_hbm,
                   src_v, dst_v, rows_v, zb_v, acc_sh):
        cid = lax.axis_index("c")
        sid = lax.axis_index("s")
        wid = sid * NC + cid

        _zero_accumulator(zb_v, acc_sh, sid, d)
        plsc.subcore_barrier()

        pltpu.sync_copy(src_hbm.at[wid], src_v)
        pltpu.sync_copy(dst_hbm.at[wid], dst_v)
        _agg_loop(table_hbm, src_v, dst_v, rows_v, acc_sh, n_chunks)

        plsc.subcore_barrier()
        pltpu.sync_copy(acc_sh.at[pl.ds(sid * SROWS, SROWS)],
                        out_hbm.at[cid, pl.ds(sid * SROWS, SROWS)])

    return agg_kernel(table, src3, dst3)


def _tc_mlp(agg, w1, b1_2d, w2p, bm=1024):
    d_half = agg.shape[2]
    d_hid = w1.shape[1]
    d_out = w2p.shape[1]

    def body(a0, a1, w1r, b1r, w2r, o):
        full = jnp.concatenate([a0[0], a1[0]], axis=1)
        h = jnp.dot(full, w1r[...], preferred_element_type=jnp.float32)
        h = jnp.maximum(h + b1r[...], 0.0)
        o[...] = jnp.dot(h, w2r[...], preferred_element_type=jnp.float32)

    return pl.pallas_call(
        body,
        grid=(NPAD // bm,),
        in_specs=[
            pl.BlockSpec((1, bm, d_half), lambda i: (0, i, 0)),
            pl.BlockSpec((1, bm, d_half), lambda i: (1, i, 0)),
            pl.BlockSpec((2 * d_half, d_hid), lambda i: (0, 0)),
            pl.BlockSpec((1, d_hid), lambda i: (0, 0)),
            pl.BlockSpec((d_hid, d_out), lambda i: (0, 0)),
        ],
        out_specs=pl.BlockSpec((bm, d_out), lambda i: (i, 0)),
        out_shape=jax.ShapeDtypeStruct((NPAD, d_out), jnp.float32),
    )(agg, agg, w1, b1_2d, w2p)


def _tc_final(acc, b2_2d, bm=1024):
    d = acc.shape[2]

    def body(a0, a1, b2r, o):
        o[...] = a0[0] + a1[0] + b2r[...]

    return pl.pallas_call(
        body,
        grid=(NPAD // bm,),
        in_specs=[
            pl.BlockSpec((1, bm, d), lambda i: (0, i, 0)),
            pl.BlockSpec((1, bm, d), lambda i: (1, i, 0)),
            pl.BlockSpec((1, d), lambda i: (0, 0)),
        ],
        out_specs=pl.BlockSpec((bm, d), lambda i: (i, 0)),
        out_shape=jax.ShapeDtypeStruct((NPAD, d), jnp.float32),
    )(acc, acc, b2_2d)


def kernel(x, edge_index, W1, b1, W2, b2):
    n_nodes, d_in = x.shape
    n_edges = edge_index.shape[1]
    d_hid = W1.shape[1]
    d_out = W2.shape[1]
    d_half = d_in // NC
    d_out_pad = 48

    ei = edge_index.astype(jnp.int32)

    nch1 = n_edges // (NS * K)
    src1 = ei[0].reshape(NS, nch1, K)
    dst1 = ei[1].reshape(NS, nch1, K)
    acc1 = _sc_agg_featsplit(x[:, :d_half], x[:, d_half:],
                             src1, dst1, nch1, d_half)

    w2p = jnp.pad(W2, ((0, 0), (0, d_out_pad - d_out)))
    g = _tc_mlp(acc1, W1, b1.reshape(1, d_hid), w2p)

    nch2 = n_edges // (NW * K)
    src2 = ei[0].reshape(NW, nch2, K)
    dst2 = ei[1].reshape(NW, nch2, K)
    acc2 = _sc_agg_edgesplit(g, src2, dst2, nch2, d_out_pad)

    b2p = jnp.pad(b2, (0, d_out_pad - d_out)).reshape(1, d_out_pad)
    out = _tc_final(acc2, b2p)
    return out[:n_nodes, :d_out]
